# baseline (speedup 1.0000x reference)
import jax
import jax.numpy as jnp
from jax import lax
from jax.experimental import pallas as pl
from jax.experimental.pallas import tpu as pltpu
from jax.experimental.pallas import tpu_sc as plsc

ROWS = 59049
BATCH = 128
SUB = 243
CH = 81
CHW = CH * BATCH
SUBW = SUB * BATCH
NCHUNKS = ROWS // CH
NC = 2
NS = 16
NW = NC * NS
KMAX = (NCHUNKS + NW - 1) // NW
LANES = 16

_MESH = plsc.VectorSubcoreMesh(core_axis_name="c", subcore_axis_name="s")


def _sc_body(x_hbm, cs_hbm, re_hbm, im_hbm, cs_v,
             in_v0, in_v1, re_v0, re_v1, im_v0, im_v1,
             sin0, sin1, sre0, sre1, sim0, sim1):
    in_v = (in_v0, in_v1)
    re_v = (re_v0, re_v1)
    im_v = (im_v0, im_v1)
    sin = (sin0, sin1)
    sre = (sre0, sre1)
    sim = (sim0, sim1)

    wid = lax.axis_index("s") * NC + lax.axis_index("c")
    pltpu.sync_copy(cs_hbm, cs_v)
    cvec = cs_v[pl.ds(0, LANES)]
    nsvec = cs_v[pl.ds(LANES, LANES)]
    onev = cs_v[pl.ds(2 * LANES, LANES)]
    zerov = cs_v[pl.ds(3 * LANES, LANES)]

    def chunk_info(j):
        k = wid + j * NW
        valid = k < NCHUNKS
        a = (k // 3) % 3
        w0 = k * CHW
        im_w0 = (w0 + SUBW * jnp.where(a == 0, 1, 0)
                 - SUBW * jnp.where(a == 1, 1, 0))
        return k, valid, a, w0, im_w0

    def start_in(j):
        _, valid, _, w0, _ = chunk_info(j)
        p = j % 2

        @pl.when(valid)
        def _():
            pltpu.make_async_copy(
                x_hbm.at[pl.ds(w0, CHW)], in_v[p], sin[p]).start()

    start_in(0)
    for j in range(KMAX):
        p = j % 2
        if j + 1 < KMAX:
            start_in(j + 1)
        _, valid, a, w0, im_w0 = chunk_info(j)

        @pl.when(valid)
        def _(j=j, p=p, a=a, w0=w0, im_w0=im_w0):
            pltpu.make_async_copy(
                x_hbm.at[pl.ds(w0, CHW)], in_v[p], sin[p]).wait()
            if j >= 2:
                pltpu.make_async_copy(
                    re_v[p], re_hbm.at[pl.ds(0, CHW)], sre[p]).wait()
                pltpu.make_async_copy(
                    im_v[p], im_hbm.at[pl.ds(0, CHW)], sim[p]).wait()
            re_scale = jnp.where(a < 2, cvec, onev)
            im_scale = jnp.where(a < 2, nsvec, zerov)

            def row_body(r, _):
                base = r * 3 * BATCH
                for c in range(3 * BATCH // LANES):
                    o = base + c * LANES
                    v = in_v[p][pl.ds(o, LANES)]
                    re_v[p][pl.ds(o, LANES)] = v * re_scale
                    im_v[p][pl.ds(o, LANES)] = v * im_scale
                return 0

            lax.fori_loop(0, CH // 3, row_body, 0)
            pltpu.make_async_copy(
                re_v[p], re_hbm.at[pl.ds(w0, CHW)], sre[p]).start()
            pltpu.make_async_copy(
                im_v[p], im_hbm.at[pl.ds(im_w0, CHW)], sim[p]).start()

    for j in (KMAX - 2, KMAX - 1):
        _, valid, _, _, _ = chunk_info(j)
        p = j % 2

        @pl.when(valid)
        def _(p=p):
            pltpu.make_async_copy(
                re_v[p], re_hbm.at[pl.ds(0, CHW)], sre[p]).wait()
            pltpu.make_async_copy(
                im_v[p], im_hbm.at[pl.ds(0, CHW)], sim[p]).wait()


def _sc_planes(x1, cs):
    run = pl.kernel(
        _sc_body,
        mesh=_MESH,
        out_type=[
            jax.ShapeDtypeStruct((ROWS * BATCH,), jnp.float32),
            jax.ShapeDtypeStruct((ROWS * BATCH,), jnp.float32),
        ],
        scratch_types=[
            pltpu.VMEM((4 * LANES,), jnp.float32),
            pltpu.VMEM((CHW,), jnp.float32),
            pltpu.VMEM((CHW,), jnp.float32),
            pltpu.VMEM((CHW,), jnp.float32),
            pltpu.VMEM((CHW,), jnp.float32),
            pltpu.VMEM((CHW,), jnp.float32),
            pltpu.VMEM((CHW,), jnp.float32),
            pltpu.SemaphoreType.DMA,
            pltpu.SemaphoreType.DMA,
            pltpu.SemaphoreType.DMA,
            pltpu.SemaphoreType.DMA,
            pltpu.SemaphoreType.DMA,
            pltpu.SemaphoreType.DMA,
        ],
    )
    return run(x1, cs)


def kernel(x, angle):
    half = 0.5 * angle[0]
    c = jnp.cos(half)
    ns = -jnp.sin(half)
    cs = jnp.concatenate([
        jnp.full((LANES,), c, jnp.float32),
        jnp.full((LANES,), ns, jnp.float32),
        jnp.ones((LANES,), jnp.float32),
        jnp.zeros((LANES,), jnp.float32),
    ])
    re, im = _sc_planes(x.reshape(ROWS * BATCH), cs)
    return jax.lax.complex(re.reshape(ROWS, BATCH), im.reshape(ROWS, BATCH))


# speedup vs baseline: 1.0126x; 1.0126x over previous
import jax
import jax.numpy as jnp
from jax import lax
from jax.experimental import pallas as pl
from jax.experimental.pallas import tpu as pltpu
from jax.experimental.pallas import tpu_sc as plsc

ROWS = 59049
BATCH = 128
SUB = 243
GROUP = 3 * SUB
BLK_GROUPS = 8
BLK = GROUP * BLK_GROUPS

CH = 81
CHW = CH * BATCH
SUBW = SUB * BATCH
NCHUNKS = ROWS // CH
NC = 2
NS = 16
NW = NC * NS
KMAX = (NCHUNKS + NW - 1) // NW
LANES = 16

_MESH = plsc.VectorSubcoreMesh(core_axis_name="c", subcore_axis_name="s")


def _re_kernel(ang_ref, x_ref, re_ref):
    c = jnp.cos(0.5 * ang_ref[0])
    for g in range(BLK_GROUPS):
        b0 = g * GROUP
        re_ref[b0:b0 + 2 * SUB, :] = c * x_ref[b0:b0 + 2 * SUB, :]
        re_ref[b0 + 2 * SUB:b0 + 3 * SUB, :] = x_ref[b0 + 2 * SUB:b0 + 3 * SUB, :]


def _re_plane(x, angle):
    return pl.pallas_call(
        _re_kernel,
        grid=(pl.cdiv(ROWS, BLK),),
        in_specs=[
            pl.BlockSpec(memory_space=pltpu.SMEM),
            pl.BlockSpec((BLK, BATCH), lambda t: (t, 0)),
        ],
        out_specs=pl.BlockSpec((BLK, BATCH), lambda t: (t, 0)),
        out_shape=jax.ShapeDtypeStruct((ROWS, BATCH), jnp.float32),
    )(angle, x)


def _sc_body(x_hbm, cs_hbm, im_hbm, cs_v,
             in_v0, in_v1, im_v0, im_v1, zero_v,
             sin0, sin1, sim0, sim1):
    in_v = (in_v0, in_v1)
    im_v = (im_v0, im_v1)
    sin = (sin0, sin1)
    sim = (sim0, sim1)

    wid = lax.axis_index("s") * NC + lax.axis_index("c")
    pltpu.sync_copy(cs_hbm, cs_v)
    nsvec = cs_v[pl.ds(LANES, LANES)]
    zerov = cs_v[pl.ds(3 * LANES, LANES)]

    def zrow(r, _):
        base = r * BATCH
        for c in range(BATCH // LANES):
            zero_v[pl.ds(base + c * LANES, LANES)] = zerov
        return 0

    lax.fori_loop(0, CH, zrow, 0)

    def chunk_info(j):
        k = wid + j * NW
        valid = k < NCHUNKS
        a = (k // 3) % 3
        w0 = k * CHW
        im_w0 = (w0 + SUBW * jnp.where(a == 0, 1, 0)
                 - SUBW * jnp.where(a == 1, 1, 0))
        return k, valid, a, w0, im_w0

    def start_in(j):
        _, valid, a, w0, _ = chunk_info(j)
        p = j % 2

        @pl.when(valid & (a < 2))
        def _():
            pltpu.make_async_copy(
                x_hbm.at[pl.ds(w0, CHW)], in_v[p], sin[p]).start()

    start_in(0)
    for j in range(KMAX):
        p = j % 2
        if j + 1 < KMAX:
            start_in(j + 1)
        _, valid, a, w0, im_w0 = chunk_info(j)

        if j >= 2:
            @pl.when(valid)
            def _(p=p):
                pltpu.make_async_copy(
                    im_v[p], im_hbm.at[pl.ds(0, CHW)], sim[p]).wait()

        @pl.when(valid & (a < 2))
        def _(p=p, w0=w0, im_w0=im_w0):
            pltpu.make_async_copy(
                x_hbm.at[pl.ds(w0, CHW)], in_v[p], sin[p]).wait()

            def row_body(r, _):
                base = r * BATCH
                for c in range(BATCH // LANES):
                    o = base + c * LANES
                    im_v[p][pl.ds(o, LANES)] = in_v[p][pl.ds(o, LANES)] * nsvec
                return 0

            lax.fori_loop(0, CH, row_body, 0)
            pltpu.make_async_copy(
                im_v[p], im_hbm.at[pl.ds(im_w0, CHW)], sim[p]).start()

        @pl.when(valid & (a == 2))
        def _(p=p, w0=w0):
            pltpu.make_async_copy(
                zero_v, im_hbm.at[pl.ds(w0, CHW)], sim[p]).start()

    for j in (KMAX - 2, KMAX - 1):
        _, valid, _, _, _ = chunk_info(j)
        p = j % 2

        @pl.when(valid)
        def _(p=p):
            pltpu.make_async_copy(
                im_v[p], im_hbm.at[pl.ds(0, CHW)], sim[p]).wait()


def _im_plane(x1, cs):
    run = pl.kernel(
        _sc_body,
        mesh=_MESH,
        out_type=jax.ShapeDtypeStruct((ROWS * BATCH,), jnp.float32),
        scratch_types=[
            pltpu.VMEM((4 * LANES,), jnp.float32),
            pltpu.VMEM((CHW,), jnp.float32),
            pltpu.VMEM((CHW,), jnp.float32),
            pltpu.VMEM((CHW,), jnp.float32),
            pltpu.VMEM((CHW,), jnp.float32),
            pltpu.VMEM((CHW,), jnp.float32),
            pltpu.SemaphoreType.DMA,
            pltpu.SemaphoreType.DMA,
            pltpu.SemaphoreType.DMA,
            pltpu.SemaphoreType.DMA,
        ],
    )
    return run(x1, cs)


def kernel(x, angle):
    half = 0.5 * angle[0]
    ns = -jnp.sin(half)
    cs = jnp.concatenate([
        jnp.zeros((LANES,), jnp.float32),
        jnp.full((LANES,), ns, jnp.float32),
        jnp.ones((LANES,), jnp.float32),
        jnp.zeros((LANES,), jnp.float32),
    ])
    im = _im_plane(x.reshape(ROWS * BATCH), cs)
    re = _re_plane(x, angle)
    return jax.lax.complex(re, im.reshape(ROWS, BATCH))
